# Initial kernel scaffold; baseline (speedup 1.0000x reference)
import jax
import jax.numpy as jnp
from jax.experimental import pallas as pl


def kernel(x, router_w, router_b, expert_w, expert_b, quant_w, quant_b):
    raise NotImplementedError("write your pallas kernel here")


# speedup vs baseline: 2.1039x; 2.1039x over previous
import jax
import jax.numpy as jnp
from jax.experimental import pallas as pl

IN_FEATS = 2048
OUT = 10
CHUNKS = 16
THRESH = 0.05
CHUNK_DIM = IN_FEATS // CHUNKS
N_TOK = 16384

GRP = 16
WIDE = CHUNKS * GRP
BLK = 512
NSTEPS = N_TOK // BLK


def _main_kernel(x_ref, w_ref, b_ref, rout_ref, outpre_ref, acts_ref):
    i = pl.program_id(0)
    lane = jax.lax.broadcasted_iota(jnp.int32, (1, WIDE), 1)
    is_logit = (lane % GRP) == (GRP - 1)
    grp = lane // GRP

    acc = jnp.dot(x_ref[:], w_ref[:], preferred_element_type=jnp.float32)
    acc = acc + b_ref[:]

    lm = jnp.where(is_logit, acc, -jnp.inf)
    m = jnp.max(lm, axis=1, keepdims=True)
    e = jnp.exp(lm - m)
    s = jnp.sum(e, axis=1, keepdims=True)
    p = e / s

    p_sel = jnp.where(is_logit, p, -1.0)
    v1 = jnp.max(p_sel, axis=1, keepdims=True)
    l1 = jnp.min(jnp.where(p_sel == v1, lane, WIDE), axis=1, keepdims=True)
    p_sel2 = jnp.where(lane == l1, -2.0, p_sel)
    v2 = jnp.max(p_sel2, axis=1, keepdims=True)
    l2 = jnp.min(jnp.where(p_sel2 == v2, lane, WIDE), axis=1, keepdims=True)

    gates = (jnp.where(grp == l1 // GRP, v1, 0.0)
             + jnp.where(grp == l2 // GRP, v2, 0.0))

    outpre_ref[:] = jnp.dot(gates * acc, rout_ref[:],
                            preferred_element_type=jnp.float32)

    colsum = jnp.sum(jnp.abs(acc), axis=0, keepdims=True)

    @pl.when(i == 0)
    def _():
        acts_ref[:] = colsum

    @pl.when(i > 0)
    def _():
        acts_ref[:] = acts_ref[:] + colsum


def _head_kernel(outpre_ref, acts_ref, rchunk_ref, qw_ref, qb_ref, o_ref):
    acts16 = jnp.dot(acts_ref[:], rchunk_ref[:],
                     preferred_element_type=jnp.float32)
    scalar_act = jnp.max(acts16) * (1.0 / (N_TOK * OUT))
    qw = qw_ref[:]
    mean_abs = jnp.sum(jnp.abs(qw)) * (1.0 / (OUT * OUT))
    wq = jnp.where(scalar_act > THRESH, qw, jnp.sign(qw) * mean_abs)
    res = jnp.dot(outpre_ref[:], wq, preferred_element_type=jnp.float32)
    res = res + qb_ref[:]
    o_ref[:] = res[:, :OUT]


def kernel(x, router_w, router_b, expert_w, expert_b, quant_w, quant_b):
    f32 = jnp.float32
    eye = jnp.eye(CHUNKS, dtype=f32)
    w_exp = expert_w[:, :, None, :] * eye[:, None, :, None]
    w_exp = jnp.pad(w_exp, ((0, 0), (0, 0), (0, 0), (0, GRP - OUT)))
    w_r = jnp.pad(router_w[:, :, None], ((0, 0), (0, 0), (GRP - 1, 0)))
    w = w_exp.reshape(IN_FEATS, WIDE) + w_r.reshape(IN_FEATS, WIDE)
    bvec = (jnp.pad(expert_b, ((0, 0), (0, GRP - OUT)))
            + jnp.pad(router_b[:, None], ((0, 0), (GRP - 1, 0))))
    bvec = bvec.reshape(1, WIDE)

    li = jnp.arange(WIDE)[:, None]
    ci = jnp.arange(CHUNKS)[None, :]
    real = (li % GRP) < OUT
    r_out = ((li % GRP == ci) & real).astype(f32)
    r_chunk = ((li // GRP == ci) & real).astype(f32)

    qw_p = jnp.pad(quant_w, ((0, GRP - OUT), (0, GRP - OUT)))
    qb_p = jnp.pad(quant_b, (0, GRP - OUT)).reshape(1, GRP)

    out_pre, acts = pl.pallas_call(
        _main_kernel,
        grid=(NSTEPS,),
        in_specs=[
            pl.BlockSpec((BLK, IN_FEATS), lambda i: (i, 0)),
            pl.BlockSpec((IN_FEATS, WIDE), lambda i: (0, 0)),
            pl.BlockSpec((1, WIDE), lambda i: (0, 0)),
            pl.BlockSpec((WIDE, CHUNKS), lambda i: (0, 0)),
        ],
        out_specs=[
            pl.BlockSpec((BLK, CHUNKS), lambda i: (i, 0)),
            pl.BlockSpec((1, WIDE), lambda i: (0, 0)),
        ],
        out_shape=[
            jax.ShapeDtypeStruct((N_TOK, CHUNKS), f32),
            jax.ShapeDtypeStruct((1, WIDE), f32),
        ],
    )(x, w, bvec, r_out)

    out = pl.pallas_call(
        _head_kernel,
        grid=(NSTEPS,),
        in_specs=[
            pl.BlockSpec((BLK, CHUNKS), lambda i: (i, 0)),
            pl.BlockSpec((1, WIDE), lambda i: (0, 0)),
            pl.BlockSpec((WIDE, CHUNKS), lambda i: (0, 0)),
            pl.BlockSpec((GRP, GRP), lambda i: (0, 0)),
            pl.BlockSpec((1, GRP), lambda i: (0, 0)),
        ],
        out_specs=pl.BlockSpec((BLK, OUT), lambda i: (i, 0)),
        out_shape=jax.ShapeDtypeStruct((N_TOK, OUT), f32),
    )(out_pre, acts, r_chunk, qw_p, qb_p)
    return out
